# Initial kernel scaffold; baseline (speedup 1.0000x reference)
import jax
import jax.numpy as jnp
from jax.experimental import pallas as pl


def kernel(input_feature, edge_index, W_l, b_l, W_r):
    raise NotImplementedError("write your pallas kernel here")


# speedup vs baseline: 5.6216x; 5.6216x over previous
import functools

import jax
import jax.numpy as jnp
from jax import lax
from jax.experimental import pallas as pl
from jax.experimental.pallas import tpu as pltpu
from jax.experimental.pallas import tpu_sc as plsc

CHUNK = 128
NC, NS = 2, 16


def _chunk_split(e, nw):
    total = e // CHUNK
    return total // nw, total % nw


def _sc_segment_sum(rows_src, src, dst, n_pad, rows_per_tile, zeros_rows):
    v, d = rows_src.shape
    e = src.shape[0]
    nw = NC * NS
    chunks_per_w, chunk_rem = _chunk_split(e, nw)

    mesh = plsc.VectorSubcoreMesh(core_axis_name="c", subcore_axis_name="s")

    @functools.partial(
        pl.kernel,
        out_type=jax.ShapeDtypeStruct((NC, n_pad, d), jnp.float32),
        mesh=mesh,
        scratch_types=[
            pltpu.VMEM((CHUNK,), jnp.int32),
            pltpu.VMEM((CHUNK,), jnp.int32),
            pltpu.VMEM((CHUNK, d), jnp.float32),
            pltpu.VMEM_SHARED((n_pad, d), jnp.float32),
            pltpu.SemaphoreType.DMA,
        ],
    )
    def agg_kernel(x_hbm, src_hbm, dst_hbm, z_hbm, out_hbm,
                   src_v, dst_v, rows_v, acc_sh, sem):
        c = lax.axis_index("c")
        s = lax.axis_index("s")
        wid = c * NS + s
        row0 = s * rows_per_tile

        pltpu.sync_copy(z_hbm, acc_sh.at[pl.ds(row0, rows_per_tile)])
        plsc.subcore_barrier()

        base = chunks_per_w * wid + jnp.minimum(wid, chunk_rem)
        n_my = chunks_per_w + (wid < chunk_rem).astype(jnp.int32)

        def chunk_body(k, carry):
            off = (base + k) * CHUNK
            pltpu.sync_copy(src_hbm.at[pl.ds(off, CHUNK)], src_v)
            pltpu.sync_copy(dst_hbm.at[pl.ds(off, CHUNK)], dst_v)
            pltpu.async_copy(x_hbm.at[src_v], rows_v, sem).wait()
            pltpu.sync_copy(rows_v, acc_sh.at[dst_v], add=True)
            return carry

        lax.fori_loop(0, n_my, chunk_body, 0)
        plsc.subcore_barrier()

        pltpu.sync_copy(acc_sh.at[pl.ds(row0, rows_per_tile)],
                        out_hbm.at[c, pl.ds(row0, rows_per_tile)])

    return agg_kernel(rows_src, src, dst, zeros_rows)


def _sc_counts(dst, n_pad, rows_per_tile, zeros_rows, d):
    e = dst.shape[0]
    nw = NC * NS
    chunks_per_w, chunk_rem = _chunk_split(e, nw)
    ones_rows = jnp.ones((CHUNK, d), jnp.float32)

    mesh = plsc.VectorSubcoreMesh(core_axis_name="c", subcore_axis_name="s")

    @functools.partial(
        pl.kernel,
        out_type=jax.ShapeDtypeStruct((NC, n_pad, d), jnp.float32),
        mesh=mesh,
        scratch_types=[
            pltpu.VMEM((CHUNK,), jnp.int32),
            pltpu.VMEM((CHUNK, d), jnp.float32),
            pltpu.VMEM_SHARED((n_pad, d), jnp.float32),
        ],
    )
    def cnt_kernel(dst_hbm, z_hbm, ones_hbm, out_hbm,
                   dst_v, ones_v, acc_sh):
        c = lax.axis_index("c")
        s = lax.axis_index("s")
        wid = c * NS + s
        row0 = s * rows_per_tile

        pltpu.sync_copy(z_hbm, acc_sh.at[pl.ds(row0, rows_per_tile)])
        pltpu.sync_copy(ones_hbm, ones_v)
        plsc.subcore_barrier()

        base = chunks_per_w * wid + jnp.minimum(wid, chunk_rem)
        n_my = chunks_per_w + (wid < chunk_rem).astype(jnp.int32)

        def chunk_body(k, carry):
            off = (base + k) * CHUNK
            pltpu.sync_copy(dst_hbm.at[pl.ds(off, CHUNK)], dst_v)
            pltpu.sync_copy(ones_v, acc_sh.at[dst_v], add=True)
            return carry

        lax.fori_loop(0, n_my, chunk_body, 0)
        plsc.subcore_barrier()

        pltpu.sync_copy(acc_sh.at[pl.ds(row0, rows_per_tile)],
                        out_hbm.at[c, pl.ds(row0, rows_per_tile)])

    return cnt_kernel(dst, zeros_rows, ones_rows)


def _tc_finish_body(agg_ref, cnt_ref, x_ref, wl_ref, wr_ref, b_ref, out_ref):
    agg = agg_ref[0] + agg_ref[1]
    cnt = cnt_ref[0, :, 0:1] + cnt_ref[1, :, 0:1]
    mean = agg / jnp.maximum(cnt, 1.0)
    dn = (((1,), (1,)), ((), ()))
    out_ref[...] = (
        lax.dot_general(mean, wl_ref[...], dn,
                        preferred_element_type=jnp.float32)
        + lax.dot_general(x_ref[...], wr_ref[...], dn,
                          preferred_element_type=jnp.float32)
        + b_ref[...]
    )


def _tc_finish(agg_part, cnt_part, x, w_l, b_l, w_r):
    n, d = x.shape
    blk = 400
    assert n % blk == 0
    grid = (n // blk,)
    return pl.pallas_call(
        _tc_finish_body,
        grid=grid,
        in_specs=[
            pl.BlockSpec((2, blk, d), lambda i: (0, i, 0)),
            pl.BlockSpec((2, blk, d), lambda i: (0, i, 0)),
            pl.BlockSpec((blk, d), lambda i: (i, 0)),
            pl.BlockSpec((d, d), lambda i: (0, 0)),
            pl.BlockSpec((d, d), lambda i: (0, 0)),
            pl.BlockSpec((1, d), lambda i: (0, 0)),
        ],
        out_specs=pl.BlockSpec((blk, d), lambda i: (i, 0)),
        out_shape=jax.ShapeDtypeStruct((n, d), jnp.float32),
    )(agg_part, cnt_part, x, w_l, w_r, b_l.reshape(1, d))


@jax.jit
def kernel(input_feature, edge_index, W_l, b_l, W_r):
    x = input_feature.astype(jnp.float32)
    src = edge_index[0].astype(jnp.int32)
    dst = edge_index[1].astype(jnp.int32)
    n, d = x.shape
    rows_per_tile = (-(-n // NS) + 7) // 8 * 8
    n_pad = rows_per_tile * NS
    zeros_rows = jnp.zeros((rows_per_tile, d), jnp.float32)
    agg_part = _sc_segment_sum(x, src, dst, n_pad, rows_per_tile, zeros_rows)
    cnt_part = _sc_counts(dst, n_pad, rows_per_tile, zeros_rows, d)
    return _tc_finish(agg_part, cnt_part, x, W_l, b_l, W_r)


# speedup vs baseline: 7.8949x; 1.4044x over previous
import functools

import jax
import jax.numpy as jnp
from jax import lax
from jax.experimental import pallas as pl
from jax.experimental.pallas import tpu as pltpu
from jax.experimental.pallas import tpu_sc as plsc

CHUNK = 128
NC, NS = 2, 16


def _chunk_split(e, nw):
    total = e // CHUNK
    return total // nw, total % nw


def _sc_segment_sum(rows_src, src, dst, n_pad, rows_per_tile, zeros_rows):
    v, d = rows_src.shape
    e = src.shape[0]
    nw = NC * NS
    chunks_per_w, chunk_rem = _chunk_split(e, nw)

    mesh = plsc.VectorSubcoreMesh(core_axis_name="c", subcore_axis_name="s")

    @functools.partial(
        pl.kernel,
        out_type=jax.ShapeDtypeStruct((NC, n_pad, d), jnp.float32),
        mesh=mesh,
        scratch_types=[
            pltpu.VMEM((CHUNK,), jnp.int32),
            pltpu.VMEM((CHUNK,), jnp.int32),
            pltpu.VMEM((CHUNK,), jnp.int32),
            pltpu.VMEM((CHUNK,), jnp.int32),
            pltpu.VMEM((CHUNK, d), jnp.float32),
            pltpu.VMEM((CHUNK, d), jnp.float32),
            pltpu.VMEM_SHARED((n_pad, d), jnp.float32),
            pltpu.SemaphoreType.DMA,
            pltpu.SemaphoreType.DMA,
            pltpu.SemaphoreType.DMA,
            pltpu.SemaphoreType.DMA,
        ],
    )
    def agg_kernel(x_hbm, src_hbm, dst_hbm, z_hbm, out_hbm,
                   src_a, src_b, dst_a, dst_b, rows_a, rows_b, acc_sh,
                   sem_ga, sem_gb, sem_ia, sem_ib):
        c = lax.axis_index("c")
        s = lax.axis_index("s")
        wid = c * NS + s
        row0 = s * rows_per_tile

        pltpu.sync_copy(z_hbm, acc_sh.at[pl.ds(row0, rows_per_tile)])
        plsc.subcore_barrier()

        base = chunks_per_w * wid + jnp.minimum(wid, chunk_rem)
        n_my = chunks_per_w + (wid < chunk_rem).astype(jnp.int32)

        def off(k):
            return (base + k) * CHUNK

        @pl.when(n_my > 0)
        def _():
            pltpu.sync_copy(src_hbm.at[pl.ds(off(0), CHUNK)], src_a)
            pltpu.sync_copy(dst_hbm.at[pl.ds(off(0), CHUNK)], dst_a)
            pltpu.async_copy(x_hbm.at[src_a], rows_a, sem_ga)

        @pl.when(n_my > 1)
        def _():
            pltpu.async_copy(src_hbm.at[pl.ds(off(1), CHUNK)], src_b, sem_ib)
            pltpu.async_copy(dst_hbm.at[pl.ds(off(1), CHUNK)], dst_b, sem_ib)

        def make_stage(cs, cd, cr, sem_g, nxs, nxd, nxr, sem_gn,
                       sem_in, sem_ic):
            def stage(k):
                pltpu.make_async_copy(x_hbm.at[cs], cr, sem_g).wait()

                @pl.when(k + 1 < n_my)
                def _():
                    pltpu.make_async_copy(
                        src_hbm.at[pl.ds(off(k + 1), CHUNK)], nxs,
                        sem_in).wait()
                    pltpu.make_async_copy(
                        dst_hbm.at[pl.ds(off(k + 1), CHUNK)], nxd,
                        sem_in).wait()
                    pltpu.async_copy(x_hbm.at[nxs], nxr, sem_gn)

                pltpu.sync_copy(cr, acc_sh.at[cd], add=True)

                @pl.when(k + 2 < n_my)
                def _():
                    pltpu.async_copy(
                        src_hbm.at[pl.ds(off(k + 2), CHUNK)], cs, sem_ic)
                    pltpu.async_copy(
                        dst_hbm.at[pl.ds(off(k + 2), CHUNK)], cd, sem_ic)
            return stage

        stage_even = make_stage(src_a, dst_a, rows_a, sem_ga,
                                src_b, dst_b, rows_b, sem_gb, sem_ib, sem_ia)
        stage_odd = make_stage(src_b, dst_b, rows_b, sem_gb,
                               src_a, dst_a, rows_a, sem_ga, sem_ia, sem_ib)

        def loop_body(k, carry):
            even = (k % 2) == 0

            @pl.when(even)
            def _():
                stage_even(k)

            @pl.when(jnp.logical_not(even))
            def _():
                stage_odd(k)

            return carry

        lax.fori_loop(0, n_my, loop_body, 0)
        plsc.subcore_barrier()

        pltpu.sync_copy(acc_sh.at[pl.ds(row0, rows_per_tile)],
                        out_hbm.at[c, pl.ds(row0, rows_per_tile)])

    return agg_kernel(rows_src, src, dst, zeros_rows)


def _sc_counts(dst, n_pad, rows_per_tile, zeros_rows, d):
    e = dst.shape[0]
    nw = NC * NS
    chunks_per_w, chunk_rem = _chunk_split(e, nw)
    ones_rows = jnp.ones((CHUNK, d), jnp.float32)

    mesh = plsc.VectorSubcoreMesh(core_axis_name="c", subcore_axis_name="s")

    @functools.partial(
        pl.kernel,
        out_type=jax.ShapeDtypeStruct((NC, n_pad, d), jnp.float32),
        mesh=mesh,
        scratch_types=[
            pltpu.VMEM((CHUNK,), jnp.int32),
            pltpu.VMEM((CHUNK,), jnp.int32),
            pltpu.VMEM((CHUNK, d), jnp.float32),
            pltpu.VMEM_SHARED((n_pad, d), jnp.float32),
            pltpu.SemaphoreType.DMA,
            pltpu.SemaphoreType.DMA,
        ],
    )
    def cnt_kernel(dst_hbm, z_hbm, ones_hbm, out_hbm,
                   dst_a, dst_b, ones_v, acc_sh, sem_ia, sem_ib):
        c = lax.axis_index("c")
        s = lax.axis_index("s")
        wid = c * NS + s
        row0 = s * rows_per_tile

        pltpu.sync_copy(z_hbm, acc_sh.at[pl.ds(row0, rows_per_tile)])
        pltpu.sync_copy(ones_hbm, ones_v)
        plsc.subcore_barrier()

        base = chunks_per_w * wid + jnp.minimum(wid, chunk_rem)
        n_my = chunks_per_w + (wid < chunk_rem).astype(jnp.int32)

        def off(k):
            return (base + k) * CHUNK

        @pl.when(n_my > 0)
        def _():
            pltpu.sync_copy(dst_hbm.at[pl.ds(off(0), CHUNK)], dst_a)

        @pl.when(n_my > 1)
        def _():
            pltpu.async_copy(dst_hbm.at[pl.ds(off(1), CHUNK)], dst_b, sem_ib)

        def make_stage(cd, sem_ic, nxd, sem_in):
            def stage(k):
                @pl.when(k + 1 < n_my)
                def _():
                    pltpu.make_async_copy(
                        dst_hbm.at[pl.ds(off(k + 1), CHUNK)], nxd,
                        sem_in).wait()

                pltpu.sync_copy(ones_v, acc_sh.at[cd], add=True)

                @pl.when(k + 2 < n_my)
                def _():
                    pltpu.async_copy(
                        dst_hbm.at[pl.ds(off(k + 2), CHUNK)], cd, sem_ic)
            return stage

        stage_even = make_stage(dst_a, sem_ia, dst_b, sem_ib)
        stage_odd = make_stage(dst_b, sem_ib, dst_a, sem_ia)

        def loop_body(k, carry):
            even = (k % 2) == 0

            @pl.when(even)
            def _():
                stage_even(k)

            @pl.when(jnp.logical_not(even))
            def _():
                stage_odd(k)

            return carry

        lax.fori_loop(0, n_my, loop_body, 0)
        plsc.subcore_barrier()

        pltpu.sync_copy(acc_sh.at[pl.ds(row0, rows_per_tile)],
                        out_hbm.at[c, pl.ds(row0, rows_per_tile)])

    return cnt_kernel(dst, zeros_rows, ones_rows)


def _tc_finish_body(agg_ref, cnt_ref, x_ref, wl_ref, wr_ref, b_ref, out_ref):
    agg = agg_ref[0] + agg_ref[1]
    cnt = cnt_ref[0, :, 0:1] + cnt_ref[1, :, 0:1]
    mean = agg / jnp.maximum(cnt, 1.0)
    dn = (((1,), (1,)), ((), ()))
    out_ref[...] = (
        lax.dot_general(mean, wl_ref[...], dn,
                        preferred_element_type=jnp.float32)
        + lax.dot_general(x_ref[...], wr_ref[...], dn,
                          preferred_element_type=jnp.float32)
        + b_ref[...]
    )


def _tc_finish(agg_part, cnt_part, x, w_l, b_l, w_r):
    n, d = x.shape
    blk = 400
    assert n % blk == 0
    grid = (n // blk,)
    return pl.pallas_call(
        _tc_finish_body,
        grid=grid,
        in_specs=[
            pl.BlockSpec((2, blk, d), lambda i: (0, i, 0)),
            pl.BlockSpec((2, blk, d), lambda i: (0, i, 0)),
            pl.BlockSpec((blk, d), lambda i: (i, 0)),
            pl.BlockSpec((d, d), lambda i: (0, 0)),
            pl.BlockSpec((d, d), lambda i: (0, 0)),
            pl.BlockSpec((1, d), lambda i: (0, 0)),
        ],
        out_specs=pl.BlockSpec((blk, d), lambda i: (i, 0)),
        out_shape=jax.ShapeDtypeStruct((n, d), jnp.float32),
    )(agg_part, cnt_part, x, w_l, w_r, b_l.reshape(1, d))


@jax.jit
def kernel(input_feature, edge_index, W_l, b_l, W_r):
    x = input_feature.astype(jnp.float32)
    src = edge_index[0].astype(jnp.int32)
    dst = edge_index[1].astype(jnp.int32)
    n, d = x.shape
    rows_per_tile = (-(-n // NS) + 7) // 8 * 8
    n_pad = rows_per_tile * NS
    zeros_rows = jnp.zeros((rows_per_tile, d), jnp.float32)
    agg_part = _sc_segment_sum(x, src, dst, n_pad, rows_per_tile, zeros_rows)
    cnt_part = _sc_counts(dst, n_pad, rows_per_tile, zeros_rows, d)
    return _tc_finish(agg_part, cnt_part, x, W_l, b_l, W_r)


# speedup vs baseline: 8.8697x; 1.1235x over previous
import functools

import jax
import jax.numpy as jnp
from jax import lax
from jax.experimental import pallas as pl
from jax.experimental.pallas import tpu as pltpu
from jax.experimental.pallas import tpu_sc as plsc

CHUNK = 128
NC, NS = 2, 16


def _chunk_split(e, nw):
    total = e // CHUNK
    return total // nw, total % nw


def _sc_segment_sum(rows_src, src, dst, n_pad, rows_per_tile, zeros_rows):
    v, d = rows_src.shape
    e = src.shape[0]
    nw = NC * NS
    chunks_per_w, chunk_rem = _chunk_split(e, nw)

    mesh = plsc.VectorSubcoreMesh(core_axis_name="c", subcore_axis_name="s")

    @functools.partial(
        pl.kernel,
        out_type=jax.ShapeDtypeStruct((NC, n_pad, d), jnp.float32),
        mesh=mesh,
        scratch_types=[
            pltpu.VMEM((CHUNK,), jnp.int32),
            pltpu.VMEM((CHUNK,), jnp.int32),
            pltpu.VMEM((CHUNK,), jnp.int32),
            pltpu.VMEM((CHUNK,), jnp.int32),
            pltpu.VMEM((CHUNK, d), jnp.float32),
            pltpu.VMEM((CHUNK, d), jnp.float32),
            pltpu.VMEM_SHARED((n_pad, d), jnp.float32),
            pltpu.SemaphoreType.DMA,
            pltpu.SemaphoreType.DMA,
            pltpu.SemaphoreType.DMA,
            pltpu.SemaphoreType.DMA,
        ],
    )
    def agg_kernel(x_hbm, src_hbm, dst_hbm, z_hbm, out_hbm,
                   src_a, src_b, dst_a, dst_b, rows_a, rows_b, acc_sh,
                   sem_ga, sem_gb, sem_ia, sem_ib):
        c = lax.axis_index("c")
        s = lax.axis_index("s")
        wid = c * NS + s
        row0 = s * rows_per_tile

        pltpu.sync_copy(z_hbm, acc_sh.at[pl.ds(row0, rows_per_tile)])
        plsc.subcore_barrier()

        base = chunks_per_w * wid + jnp.minimum(wid, chunk_rem)
        n_my = chunks_per_w + (wid < chunk_rem).astype(jnp.int32)

        def off(k):
            return (base + k) * CHUNK

        @pl.when(n_my > 0)
        def _():
            pltpu.sync_copy(src_hbm.at[pl.ds(off(0), CHUNK)], src_a)
            pltpu.sync_copy(dst_hbm.at[pl.ds(off(0), CHUNK)], dst_a)
            pltpu.async_copy(x_hbm.at[src_a], rows_a, sem_ga)

        @pl.when(n_my > 1)
        def _():
            pltpu.async_copy(src_hbm.at[pl.ds(off(1), CHUNK)], src_b, sem_ib)
            pltpu.async_copy(dst_hbm.at[pl.ds(off(1), CHUNK)], dst_b, sem_ib)

        def make_stage(cs, cd, cr, sem_g, nxs, nxd, nxr, sem_gn,
                       sem_in, sem_ic):
            def stage(k):
                pltpu.make_async_copy(x_hbm.at[cs], cr, sem_g).wait()

                @pl.when(k + 1 < n_my)
                def _():
                    pltpu.make_async_copy(
                        src_hbm.at[pl.ds(off(k + 1), CHUNK)], nxs,
                        sem_in).wait()
                    pltpu.make_async_copy(
                        dst_hbm.at[pl.ds(off(k + 1), CHUNK)], nxd,
                        sem_in).wait()
                    pltpu.async_copy(x_hbm.at[nxs], nxr, sem_gn)

                pltpu.sync_copy(cr, acc_sh.at[cd], add=True)

                @pl.when(k + 2 < n_my)
                def _():
                    pltpu.async_copy(
                        src_hbm.at[pl.ds(off(k + 2), CHUNK)], cs, sem_ic)
                    pltpu.async_copy(
                        dst_hbm.at[pl.ds(off(k + 2), CHUNK)], cd, sem_ic)
            return stage

        stage_even = make_stage(src_a, dst_a, rows_a, sem_ga,
                                src_b, dst_b, rows_b, sem_gb, sem_ib, sem_ia)
        stage_odd = make_stage(src_b, dst_b, rows_b, sem_gb,
                               src_a, dst_a, rows_a, sem_ga, sem_ia, sem_ib)

        def loop_body(k, carry):
            even = (k % 2) == 0

            @pl.when(even)
            def _():
                stage_even(k)

            @pl.when(jnp.logical_not(even))
            def _():
                stage_odd(k)

            return carry

        lax.fori_loop(0, n_my, loop_body, 0)
        plsc.subcore_barrier()

        pltpu.sync_copy(acc_sh.at[pl.ds(row0, rows_per_tile)],
                        out_hbm.at[c, pl.ds(row0, rows_per_tile)])

    return agg_kernel(rows_src, src, dst, zeros_rows)


def _sc_counts(dst, n_pad, rows_per_tile, zeros_rows, d):
    e = dst.shape[0]
    nw = NC * NS
    chunks_per_w, chunk_rem = _chunk_split(e, nw)
    ones_rows = jnp.ones((CHUNK, d), jnp.float32)

    mesh = plsc.VectorSubcoreMesh(core_axis_name="c", subcore_axis_name="s")

    nb = chunks_per_w
    assert nb >= 3

    @functools.partial(
        pl.kernel,
        out_type=jax.ShapeDtypeStruct((NC, n_pad, d), jnp.float32),
        mesh=mesh,
        scratch_types=[
            pltpu.VMEM((CHUNK,), jnp.int32),
            pltpu.VMEM((CHUNK,), jnp.int32),
            pltpu.VMEM((CHUNK,), jnp.int32),
            pltpu.VMEM((CHUNK, d), jnp.float32),
            pltpu.VMEM_SHARED((n_pad, d), jnp.float32),
            pltpu.SemaphoreType.DMA,
            pltpu.SemaphoreType.DMA,
            pltpu.SemaphoreType.DMA,
            pltpu.SemaphoreType.DMA,
            pltpu.SemaphoreType.DMA,
            pltpu.SemaphoreType.DMA,
        ],
    )
    def cnt_kernel(dst_hbm, z_hbm, ones_hbm, out_hbm,
                   d0, d1, d2, ones_v, acc_sh,
                   si0, si1, si2, ss0, ss1, ss2):
        c = lax.axis_index("c")
        s = lax.axis_index("s")
        wid = c * NS + s
        row0 = s * rows_per_tile

        pltpu.sync_copy(z_hbm, acc_sh.at[pl.ds(row0, rows_per_tile)])
        pltpu.sync_copy(ones_hbm, ones_v)
        plsc.subcore_barrier()

        base = wid * nb

        def off(k):
            return (base + k) * CHUNK

        dbufs = (d0, d1, d2)
        isems = (si0, si1, si2)
        ssems = (ss0, ss1, ss2)

        pltpu.sync_copy(dst_hbm.at[pl.ds(off(0), CHUNK)], d0)
        pltpu.async_copy(dst_hbm.at[pl.ds(off(1), CHUNK)], d1, si1)
        pltpu.async_copy(dst_hbm.at[pl.ds(off(2), CHUNK)], d2, si2)

        def make_stage(m):
            cd, semi, sems = dbufs[m], isems[m], ssems[m]
            pd, psems = dbufs[(m + 2) % 3], ssems[(m + 2) % 3]
            nd, nsemi = dbufs[(m + 2) % 3], isems[(m + 2) % 3]

            def stage(k):
                @pl.when(k >= 1)
                def _():
                    pltpu.make_async_copy(ones_v, acc_sh.at[pd], psems).wait()

                @pl.when(k >= 1)
                def _():
                    pltpu.make_async_copy(
                        dst_hbm.at[pl.ds(off(k), CHUNK)], cd, semi).wait()

                pltpu.async_copy(ones_v, acc_sh.at[cd], sems, add=True)

                @pl.when(jnp.logical_and(k >= 1, k + 2 < nb))
                def _():
                    pltpu.async_copy(
                        dst_hbm.at[pl.ds(off(k + 2), CHUNK)], nd, nsemi)
            return stage

        stages = [make_stage(m) for m in range(3)]

        def loop_body(k, carry):
            m = k % 3

            @pl.when(m == 0)
            def _():
                stages[0](k)

            @pl.when(m == 1)
            def _():
                stages[1](k)

            @pl.when(m == 2)
            def _():
                stages[2](k)

            return carry

        lax.fori_loop(0, nb, loop_body, 0)
        last = dbufs[(nb - 1) % 3]
        pltpu.make_async_copy(ones_v, acc_sh.at[last],
                              ssems[(nb - 1) % 3]).wait()

        @pl.when(wid < chunk_rem)
        def _():
            pltpu.sync_copy(dst_hbm.at[pl.ds((nw * nb + wid) * CHUNK, CHUNK)],
                            d0)
            pltpu.sync_copy(ones_v, acc_sh.at[d0], add=True)

        plsc.subcore_barrier()
        pltpu.sync_copy(acc_sh.at[pl.ds(row0, rows_per_tile)],
                        out_hbm.at[c, pl.ds(row0, rows_per_tile)])

    return cnt_kernel(dst, zeros_rows, ones_rows)


def _tc_finish_body(agg_ref, cnt_ref, x_ref, wl_ref, wr_ref, b_ref, out_ref):
    agg = agg_ref[0] + agg_ref[1]
    cnt = cnt_ref[0, :, 0:1] + cnt_ref[1, :, 0:1]
    mean = agg / jnp.maximum(cnt, 1.0)
    dn = (((1,), (1,)), ((), ()))
    out_ref[...] = (
        lax.dot_general(mean, wl_ref[...], dn,
                        preferred_element_type=jnp.float32)
        + lax.dot_general(x_ref[...], wr_ref[...], dn,
                          preferred_element_type=jnp.float32)
        + b_ref[...]
    )


def _tc_finish(agg_part, cnt_part, x, w_l, b_l, w_r):
    n, d = x.shape
    blk = 400
    assert n % blk == 0
    grid = (n // blk,)
    return pl.pallas_call(
        _tc_finish_body,
        grid=grid,
        in_specs=[
            pl.BlockSpec((2, blk, d), lambda i: (0, i, 0)),
            pl.BlockSpec((2, blk, d), lambda i: (0, i, 0)),
            pl.BlockSpec((blk, d), lambda i: (i, 0)),
            pl.BlockSpec((d, d), lambda i: (0, 0)),
            pl.BlockSpec((d, d), lambda i: (0, 0)),
            pl.BlockSpec((1, d), lambda i: (0, 0)),
        ],
        out_specs=pl.BlockSpec((blk, d), lambda i: (i, 0)),
        out_shape=jax.ShapeDtypeStruct((n, d), jnp.float32),
    )(agg_part, cnt_part, x, w_l, w_r, b_l.reshape(1, d))


@jax.jit
def kernel(input_feature, edge_index, W_l, b_l, W_r):
    x = input_feature.astype(jnp.float32)
    src = edge_index[0].astype(jnp.int32)
    dst = edge_index[1].astype(jnp.int32)
    n, d = x.shape
    rows_per_tile = (-(-n // NS) + 7) // 8 * 8
    n_pad = rows_per_tile * NS
    zeros_rows = jnp.zeros((rows_per_tile, d), jnp.float32)
    agg_part = _sc_segment_sum(x, src, dst, n_pad, rows_per_tile, zeros_rows)
    cnt_part = _sc_counts(dst, n_pad, rows_per_tile, zeros_rows, d)
    return _tc_finish(agg_part, cnt_part, x, W_l, b_l, W_r)


# speedup vs baseline: 9.6440x; 1.0873x over previous
import functools

import jax
import jax.numpy as jnp
from jax import lax
from jax.experimental import pallas as pl
from jax.experimental.pallas import tpu as pltpu
from jax.experimental.pallas import tpu_sc as plsc

CHUNK = 128
NC, NS = 2, 16


def _chunk_split(e, nw):
    total = e // CHUNK
    return total // nw, total % nw


def _sc_segment_sum(rows_src, src, dst, n_pad, rows_per_tile, zeros_rows):
    v, d = rows_src.shape
    e = src.shape[0]
    nw = NC * NS
    chunks_per_w, chunk_rem = _chunk_split(e, nw)

    mesh = plsc.VectorSubcoreMesh(core_axis_name="c", subcore_axis_name="s")
    nb = chunks_per_w
    assert nb >= 3

    @functools.partial(
        pl.kernel,
        out_type=jax.ShapeDtypeStruct((NC, n_pad, d), jnp.float32),
        mesh=mesh,
        scratch_types=(
            [pltpu.VMEM((CHUNK,), jnp.int32)] * 6
            + [pltpu.VMEM((CHUNK, d), jnp.float32)] * 3
            + [pltpu.VMEM_SHARED((n_pad, d), jnp.float32)]
            + [pltpu.SemaphoreType.DMA] * 9
        ),
    )
    def agg_kernel(x_hbm, src_hbm, dst_hbm, z_hbm, out_hbm,
                   s0, s1, s2, d0, d1, d2, r0, r1, r2, acc_sh,
                   si0, si1, si2, sg0, sg1, sg2, ss0, ss1, ss2):
        c = lax.axis_index("c")
        s = lax.axis_index("s")
        wid = c * NS + s
        row0 = s * rows_per_tile

        pltpu.sync_copy(z_hbm, acc_sh.at[pl.ds(row0, rows_per_tile)])
        plsc.subcore_barrier()

        base = wid * nb

        def off(k):
            return (base + k) * CHUNK

        sbufs = (s0, s1, s2)
        dbufs = (d0, d1, d2)
        rbufs = (r0, r1, r2)
        isems = (si0, si1, si2)
        gsems = (sg0, sg1, sg2)
        ssems = (ss0, ss1, ss2)

        pltpu.sync_copy(src_hbm.at[pl.ds(off(0), CHUNK)], s0)
        pltpu.sync_copy(dst_hbm.at[pl.ds(off(0), CHUNK)], d0)
        pltpu.async_copy(x_hbm.at[s0], r0, sg0)
        pltpu.async_copy(src_hbm.at[pl.ds(off(1), CHUNK)], s1, si1)
        pltpu.async_copy(dst_hbm.at[pl.ds(off(1), CHUNK)], d1, si1)
        pltpu.async_copy(src_hbm.at[pl.ds(off(2), CHUNK)], s2, si2)
        pltpu.async_copy(dst_hbm.at[pl.ds(off(2), CHUNK)], d2, si2)

        def make_stage(m):
            n1, n2 = (m + 1) % 3, (m + 2) % 3

            def stage(k):
                @pl.when(k >= 1)
                def _():
                    pltpu.make_async_copy(
                        rbufs[n2], acc_sh.at[dbufs[n2]], ssems[n2]).wait()

                @pl.when(k + 1 < nb)
                def _():
                    pltpu.make_async_copy(
                        src_hbm.at[pl.ds(off(k + 1), CHUNK)], sbufs[n1],
                        isems[n1]).wait()
                    pltpu.make_async_copy(
                        dst_hbm.at[pl.ds(off(k + 1), CHUNK)], dbufs[n1],
                        isems[n1]).wait()
                    pltpu.async_copy(x_hbm.at[sbufs[n1]], rbufs[n1],
                                     gsems[n1])

                pltpu.make_async_copy(x_hbm.at[sbufs[m]], rbufs[m],
                                      gsems[m]).wait()
                pltpu.async_copy(rbufs[m], acc_sh.at[dbufs[m]], ssems[m],
                                 add=True)

                @pl.when(jnp.logical_and(k >= 1, k + 2 < nb))
                def _():
                    pltpu.async_copy(
                        src_hbm.at[pl.ds(off(k + 2), CHUNK)], sbufs[n2],
                        isems[n2])
                    pltpu.async_copy(
                        dst_hbm.at[pl.ds(off(k + 2), CHUNK)], dbufs[n2],
                        isems[n2])
            return stage

        stages = [make_stage(m) for m in range(3)]

        def loop_body(k, carry):
            m = k % 3

            @pl.when(m == 0)
            def _():
                stages[0](k)

            @pl.when(m == 1)
            def _():
                stages[1](k)

            @pl.when(m == 2)
            def _():
                stages[2](k)

            return carry

        lax.fori_loop(0, nb, loop_body, 0)
        mlast = (nb - 1) % 3
        pltpu.make_async_copy(rbufs[mlast], acc_sh.at[dbufs[mlast]],
                              ssems[mlast]).wait()

        @pl.when(wid < chunk_rem)
        def _():
            koff = (nw * nb + wid) * CHUNK
            pltpu.sync_copy(src_hbm.at[pl.ds(koff, CHUNK)], s0)
            pltpu.sync_copy(dst_hbm.at[pl.ds(koff, CHUNK)], d0)
            pltpu.async_copy(x_hbm.at[s0], r0, sg0).wait()
            pltpu.sync_copy(r0, acc_sh.at[d0], add=True)

        plsc.subcore_barrier()
        pltpu.sync_copy(acc_sh.at[pl.ds(row0, rows_per_tile)],
                        out_hbm.at[c, pl.ds(row0, rows_per_tile)])

    return agg_kernel(rows_src, src, dst, zeros_rows)


def _sc_counts(dst, n_pad, rows_per_tile, zeros_rows, d):
    e = dst.shape[0]
    nw = NC * NS
    chunks_per_w, chunk_rem = _chunk_split(e, nw)
    ones_rows = jnp.ones((CHUNK, d), jnp.float32)

    mesh = plsc.VectorSubcoreMesh(core_axis_name="c", subcore_axis_name="s")

    nb = chunks_per_w
    assert nb >= 3

    @functools.partial(
        pl.kernel,
        out_type=jax.ShapeDtypeStruct((NC, n_pad, d), jnp.float32),
        mesh=mesh,
        scratch_types=[
            pltpu.VMEM((CHUNK,), jnp.int32),
            pltpu.VMEM((CHUNK,), jnp.int32),
            pltpu.VMEM((CHUNK,), jnp.int32),
            pltpu.VMEM((CHUNK, d), jnp.float32),
            pltpu.VMEM_SHARED((n_pad, d), jnp.float32),
            pltpu.SemaphoreType.DMA,
            pltpu.SemaphoreType.DMA,
            pltpu.SemaphoreType.DMA,
            pltpu.SemaphoreType.DMA,
            pltpu.SemaphoreType.DMA,
            pltpu.SemaphoreType.DMA,
        ],
    )
    def cnt_kernel(dst_hbm, z_hbm, ones_hbm, out_hbm,
                   d0, d1, d2, ones_v, acc_sh,
                   si0, si1, si2, ss0, ss1, ss2):
        c = lax.axis_index("c")
        s = lax.axis_index("s")
        wid = c * NS + s
        row0 = s * rows_per_tile

        pltpu.sync_copy(z_hbm, acc_sh.at[pl.ds(row0, rows_per_tile)])
        pltpu.sync_copy(ones_hbm, ones_v)
        plsc.subcore_barrier()

        base = wid * nb

        def off(k):
            return (base + k) * CHUNK

        dbufs = (d0, d1, d2)
        isems = (si0, si1, si2)
        ssems = (ss0, ss1, ss2)

        pltpu.sync_copy(dst_hbm.at[pl.ds(off(0), CHUNK)], d0)
        pltpu.async_copy(dst_hbm.at[pl.ds(off(1), CHUNK)], d1, si1)
        pltpu.async_copy(dst_hbm.at[pl.ds(off(2), CHUNK)], d2, si2)

        def make_stage(m):
            cd, semi, sems = dbufs[m], isems[m], ssems[m]
            pd, psems = dbufs[(m + 2) % 3], ssems[(m + 2) % 3]
            nd, nsemi = dbufs[(m + 2) % 3], isems[(m + 2) % 3]

            def stage(k):
                @pl.when(k >= 1)
                def _():
                    pltpu.make_async_copy(ones_v, acc_sh.at[pd], psems).wait()

                @pl.when(k >= 1)
                def _():
                    pltpu.make_async_copy(
                        dst_hbm.at[pl.ds(off(k), CHUNK)], cd, semi).wait()

                pltpu.async_copy(ones_v, acc_sh.at[cd], sems, add=True)

                @pl.when(jnp.logical_and(k >= 1, k + 2 < nb))
                def _():
                    pltpu.async_copy(
                        dst_hbm.at[pl.ds(off(k + 2), CHUNK)], nd, nsemi)
            return stage

        stages = [make_stage(m) for m in range(3)]

        def loop_body(k, carry):
            m = k % 3

            @pl.when(m == 0)
            def _():
                stages[0](k)

            @pl.when(m == 1)
            def _():
                stages[1](k)

            @pl.when(m == 2)
            def _():
                stages[2](k)

            return carry

        lax.fori_loop(0, nb, loop_body, 0)
        last = dbufs[(nb - 1) % 3]
        pltpu.make_async_copy(ones_v, acc_sh.at[last],
                              ssems[(nb - 1) % 3]).wait()

        @pl.when(wid < chunk_rem)
        def _():
            pltpu.sync_copy(dst_hbm.at[pl.ds((nw * nb + wid) * CHUNK, CHUNK)],
                            d0)
            pltpu.sync_copy(ones_v, acc_sh.at[d0], add=True)

        plsc.subcore_barrier()
        pltpu.sync_copy(acc_sh.at[pl.ds(row0, rows_per_tile)],
                        out_hbm.at[c, pl.ds(row0, rows_per_tile)])

    return cnt_kernel(dst, zeros_rows, ones_rows)


def _tc_finish_body(agg_ref, cnt_ref, x_ref, wl_ref, wr_ref, b_ref, out_ref):
    agg = agg_ref[0] + agg_ref[1]
    cnt = cnt_ref[0, :, 0:1] + cnt_ref[1, :, 0:1]
    mean = agg / jnp.maximum(cnt, 1.0)
    dn = (((1,), (1,)), ((), ()))
    out_ref[...] = (
        lax.dot_general(mean, wl_ref[...], dn,
                        preferred_element_type=jnp.float32)
        + lax.dot_general(x_ref[...], wr_ref[...], dn,
                          preferred_element_type=jnp.float32)
        + b_ref[...]
    )


def _tc_finish(agg_part, cnt_part, x, w_l, b_l, w_r):
    n, d = x.shape
    blk = 400
    assert n % blk == 0
    grid = (n // blk,)
    return pl.pallas_call(
        _tc_finish_body,
        grid=grid,
        in_specs=[
            pl.BlockSpec((2, blk, d), lambda i: (0, i, 0)),
            pl.BlockSpec((2, blk, d), lambda i: (0, i, 0)),
            pl.BlockSpec((blk, d), lambda i: (i, 0)),
            pl.BlockSpec((d, d), lambda i: (0, 0)),
            pl.BlockSpec((d, d), lambda i: (0, 0)),
            pl.BlockSpec((1, d), lambda i: (0, 0)),
        ],
        out_specs=pl.BlockSpec((blk, d), lambda i: (i, 0)),
        out_shape=jax.ShapeDtypeStruct((n, d), jnp.float32),
    )(agg_part, cnt_part, x, w_l, w_r, b_l.reshape(1, d))


@jax.jit
def kernel(input_feature, edge_index, W_l, b_l, W_r):
    x = input_feature.astype(jnp.float32)
    src = edge_index[0].astype(jnp.int32)
    dst = edge_index[1].astype(jnp.int32)
    n, d = x.shape
    rows_per_tile = (-(-n // NS) + 7) // 8 * 8
    n_pad = rows_per_tile * NS
    zeros_rows = jnp.zeros((rows_per_tile, d), jnp.float32)
    agg_part = _sc_segment_sum(x, src, dst, n_pad, rows_per_tile, zeros_rows)
    cnt_part = _sc_counts(dst, n_pad, rows_per_tile, zeros_rows, d)
    return _tc_finish(agg_part, cnt_part, x, W_l, b_l, W_r)


# speedup vs baseline: 9.6824x; 1.0040x over previous
import functools

import jax
import jax.numpy as jnp
from jax import lax
from jax.experimental import pallas as pl
from jax.experimental.pallas import tpu as pltpu
from jax.experimental.pallas import tpu_sc as plsc

CHUNK = 128
NC, NS = 2, 16


def _chunk_split(e, nw):
    total = e // CHUNK
    return total // nw, total % nw


def _sc_segment_sum(rows_src, src, dst, n_pad, rows_per_tile, zeros_rows):
    v, d = rows_src.shape
    e = src.shape[0]
    nw = NC * NS
    chunks_per_w, chunk_rem = _chunk_split(e, nw)

    mesh = plsc.VectorSubcoreMesh(core_axis_name="c", subcore_axis_name="s")
    nb = chunks_per_w
    assert nb >= 3

    @functools.partial(
        pl.kernel,
        out_type=jax.ShapeDtypeStruct((NC, n_pad, d), jnp.float32),
        mesh=mesh,
        scratch_types=(
            [pltpu.VMEM((CHUNK,), jnp.int32)] * 6
            + [pltpu.VMEM((CHUNK, d), jnp.float32)] * 3
            + [pltpu.VMEM_SHARED((n_pad, d), jnp.float32)]
            + [pltpu.SemaphoreType.DMA] * 9
        ),
    )
    def agg_kernel(x_hbm, src_hbm, dst_hbm, z_hbm, out_hbm,
                   s0, s1, s2, d0, d1, d2, r0, r1, r2, acc_sh,
                   si0, si1, si2, sg0, sg1, sg2, ss0, ss1, ss2):
        c = lax.axis_index("c")
        s = lax.axis_index("s")
        wid = c * NS + s
        row0 = s * rows_per_tile

        pltpu.sync_copy(z_hbm, acc_sh.at[pl.ds(row0, rows_per_tile)])

        base = wid * nb

        def off(k):
            return (base + k) * CHUNK

        sbufs = (s0, s1, s2)
        dbufs = (d0, d1, d2)
        rbufs = (r0, r1, r2)
        isems = (si0, si1, si2)
        gsems = (sg0, sg1, sg2)
        ssems = (ss0, ss1, ss2)

        pltpu.sync_copy(src_hbm.at[pl.ds(off(0), CHUNK)], s0)
        pltpu.sync_copy(dst_hbm.at[pl.ds(off(0), CHUNK)], d0)
        pltpu.async_copy(x_hbm.at[s0], r0, sg0)
        pltpu.async_copy(src_hbm.at[pl.ds(off(1), CHUNK)], s1, si1)
        pltpu.async_copy(dst_hbm.at[pl.ds(off(1), CHUNK)], d1, si1)
        pltpu.async_copy(src_hbm.at[pl.ds(off(2), CHUNK)], s2, si2)
        pltpu.async_copy(dst_hbm.at[pl.ds(off(2), CHUNK)], d2, si2)
        plsc.subcore_barrier()

        def make_stage(m):
            n1, n2 = (m + 1) % 3, (m + 2) % 3

            def stage(k):
                @pl.when(k + 1 < nb)
                def _():
                    pltpu.make_async_copy(
                        src_hbm.at[pl.ds(off(k + 1), CHUNK)], sbufs[n1],
                        isems[n1]).wait()
                    pltpu.make_async_copy(
                        dst_hbm.at[pl.ds(off(k + 1), CHUNK)], dbufs[n1],
                        isems[n1]).wait()
                    pltpu.async_copy(x_hbm.at[sbufs[n1]], rbufs[n1],
                                     gsems[n1])

                @pl.when(k >= 1)
                def _():
                    pltpu.make_async_copy(
                        rbufs[n2], acc_sh.at[dbufs[n2]], ssems[n2]).wait()

                pltpu.make_async_copy(x_hbm.at[sbufs[m]], rbufs[m],
                                      gsems[m]).wait()
                pltpu.async_copy(rbufs[m], acc_sh.at[dbufs[m]], ssems[m],
                                 add=True)

                @pl.when(jnp.logical_and(k >= 1, k + 2 < nb))
                def _():
                    pltpu.async_copy(
                        src_hbm.at[pl.ds(off(k + 2), CHUNK)], sbufs[n2],
                        isems[n2])
                    pltpu.async_copy(
                        dst_hbm.at[pl.ds(off(k + 2), CHUNK)], dbufs[n2],
                        isems[n2])
            return stage

        stages = [make_stage(m) for m in range(3)]

        def loop_body(k, carry):
            m = k % 3

            @pl.when(m == 0)
            def _():
                stages[0](k)

            @pl.when(m == 1)
            def _():
                stages[1](k)

            @pl.when(m == 2)
            def _():
                stages[2](k)

            return carry

        lax.fori_loop(0, nb, loop_body, 0)
        mlast = (nb - 1) % 3
        pltpu.make_async_copy(rbufs[mlast], acc_sh.at[dbufs[mlast]],
                              ssems[mlast]).wait()

        @pl.when(wid < chunk_rem)
        def _():
            koff = (nw * nb + wid) * CHUNK
            pltpu.sync_copy(src_hbm.at[pl.ds(koff, CHUNK)], s0)
            pltpu.sync_copy(dst_hbm.at[pl.ds(koff, CHUNK)], d0)
            pltpu.async_copy(x_hbm.at[s0], r0, sg0).wait()
            pltpu.sync_copy(r0, acc_sh.at[d0], add=True)

        plsc.subcore_barrier()
        pltpu.sync_copy(acc_sh.at[pl.ds(row0, rows_per_tile)],
                        out_hbm.at[c, pl.ds(row0, rows_per_tile)])

    return agg_kernel(rows_src, src, dst, zeros_rows)


def _sc_counts(dst, n_pad, rows_per_tile, zeros_rows, d):
    e = dst.shape[0]
    nw = NC * NS
    chunks_per_w, chunk_rem = _chunk_split(e, nw)
    ones_rows = jnp.ones((CHUNK, d), jnp.float32)

    mesh = plsc.VectorSubcoreMesh(core_axis_name="c", subcore_axis_name="s")

    nb = chunks_per_w
    assert nb >= 3

    @functools.partial(
        pl.kernel,
        out_type=jax.ShapeDtypeStruct((NC, n_pad, d), jnp.float32),
        mesh=mesh,
        scratch_types=[
            pltpu.VMEM((CHUNK,), jnp.int32),
            pltpu.VMEM((CHUNK,), jnp.int32),
            pltpu.VMEM((CHUNK,), jnp.int32),
            pltpu.VMEM((CHUNK, d), jnp.float32),
            pltpu.VMEM_SHARED((n_pad, d), jnp.float32),
            pltpu.SemaphoreType.DMA,
            pltpu.SemaphoreType.DMA,
            pltpu.SemaphoreType.DMA,
            pltpu.SemaphoreType.DMA,
            pltpu.SemaphoreType.DMA,
            pltpu.SemaphoreType.DMA,
        ],
    )
    def cnt_kernel(dst_hbm, z_hbm, ones_hbm, out_hbm,
                   d0, d1, d2, ones_v, acc_sh,
                   si0, si1, si2, ss0, ss1, ss2):
        c = lax.axis_index("c")
        s = lax.axis_index("s")
        wid = c * NS + s
        row0 = s * rows_per_tile

        pltpu.sync_copy(z_hbm, acc_sh.at[pl.ds(row0, rows_per_tile)])
        pltpu.sync_copy(ones_hbm, ones_v)

        base = wid * nb

        def off(k):
            return (base + k) * CHUNK

        dbufs = (d0, d1, d2)
        isems = (si0, si1, si2)
        ssems = (ss0, ss1, ss2)

        pltpu.sync_copy(dst_hbm.at[pl.ds(off(0), CHUNK)], d0)
        pltpu.async_copy(dst_hbm.at[pl.ds(off(1), CHUNK)], d1, si1)
        pltpu.async_copy(dst_hbm.at[pl.ds(off(2), CHUNK)], d2, si2)
        plsc.subcore_barrier()

        def make_stage(m):
            cd, semi, sems = dbufs[m], isems[m], ssems[m]
            pd, psems = dbufs[(m + 2) % 3], ssems[(m + 2) % 3]
            nd, nsemi = dbufs[(m + 2) % 3], isems[(m + 2) % 3]

            def stage(k):
                @pl.when(k >= 1)
                def _():
                    pltpu.make_async_copy(ones_v, acc_sh.at[pd], psems).wait()

                @pl.when(k >= 1)
                def _():
                    pltpu.make_async_copy(
                        dst_hbm.at[pl.ds(off(k), CHUNK)], cd, semi).wait()

                pltpu.async_copy(ones_v, acc_sh.at[cd], sems, add=True)

                @pl.when(jnp.logical_and(k >= 1, k + 2 < nb))
                def _():
                    pltpu.async_copy(
                        dst_hbm.at[pl.ds(off(k + 2), CHUNK)], nd, nsemi)
            return stage

        stages = [make_stage(m) for m in range(3)]

        def loop_body(k, carry):
            m = k % 3

            @pl.when(m == 0)
            def _():
                stages[0](k)

            @pl.when(m == 1)
            def _():
                stages[1](k)

            @pl.when(m == 2)
            def _():
                stages[2](k)

            return carry

        lax.fori_loop(0, nb, loop_body, 0)
        last = dbufs[(nb - 1) % 3]
        pltpu.make_async_copy(ones_v, acc_sh.at[last],
                              ssems[(nb - 1) % 3]).wait()

        @pl.when(wid < chunk_rem)
        def _():
            pltpu.sync_copy(dst_hbm.at[pl.ds((nw * nb + wid) * CHUNK, CHUNK)],
                            d0)
            pltpu.sync_copy(ones_v, acc_sh.at[d0], add=True)

        plsc.subcore_barrier()
        pltpu.sync_copy(acc_sh.at[pl.ds(row0, rows_per_tile)],
                        out_hbm.at[c, pl.ds(row0, rows_per_tile)])

    return cnt_kernel(dst, zeros_rows, ones_rows)


def _tc_finish_body(agg_ref, cnt_ref, x_ref, wl_ref, wr_ref, b_ref, out_ref):
    agg = agg_ref[0] + agg_ref[1]
    cnt = cnt_ref[0, :, 0:1] + cnt_ref[1, :, 0:1]
    mean = agg / jnp.maximum(cnt, 1.0)
    dn = (((1,), (1,)), ((), ()))
    out_ref[...] = (
        lax.dot_general(mean, wl_ref[...], dn,
                        preferred_element_type=jnp.float32)
        + lax.dot_general(x_ref[...], wr_ref[...], dn,
                          preferred_element_type=jnp.float32)
        + b_ref[...]
    )


def _tc_finish(agg_part, cnt_part, x, w_l, b_l, w_r):
    n, d = x.shape
    blk = 400
    assert n % blk == 0
    grid = (n // blk,)
    return pl.pallas_call(
        _tc_finish_body,
        grid=grid,
        in_specs=[
            pl.BlockSpec((2, blk, d), lambda i: (0, i, 0)),
            pl.BlockSpec((2, blk, d), lambda i: (0, i, 0)),
            pl.BlockSpec((blk, d), lambda i: (i, 0)),
            pl.BlockSpec((d, d), lambda i: (0, 0)),
            pl.BlockSpec((d, d), lambda i: (0, 0)),
            pl.BlockSpec((1, d), lambda i: (0, 0)),
        ],
        out_specs=pl.BlockSpec((blk, d), lambda i: (i, 0)),
        out_shape=jax.ShapeDtypeStruct((n, d), jnp.float32),
    )(agg_part, cnt_part, x, w_l, w_r, b_l.reshape(1, d))


@jax.jit
def kernel(input_feature, edge_index, W_l, b_l, W_r):
    x = input_feature.astype(jnp.float32)
    src = edge_index[0].astype(jnp.int32)
    dst = edge_index[1].astype(jnp.int32)
    n, d = x.shape
    rows_per_tile = (-(-n // NS) + 7) // 8 * 8
    n_pad = rows_per_tile * NS
    zeros_rows = jnp.zeros((rows_per_tile, d), jnp.float32)
    agg_part = _sc_segment_sum(x, src, dst, n_pad, rows_per_tile, zeros_rows)
    cnt_part = _sc_counts(dst, n_pad, rows_per_tile, zeros_rows, d)
    return _tc_finish(agg_part, cnt_part, x, W_l, b_l, W_r)
